# baseline (speedup 1.0000x reference)
import functools

import jax
import jax.numpy as jnp
from jax import lax
from jax.experimental import pallas as pl
from jax.experimental.pallas import tpu as pltpu
from jax.experimental.pallas import tpu_sc as plsc

_N = 10000
_D = 128
_E = 320000
_L = 16

_NC = 2
_NS = 16
_NW = _NC * _NS
_CH = 32
_T = 4992
_TSZ = 5008
_NP = 10008
_ARPT = 624
_EPAD = _E + 1536


def _seg_sum_shard(h, srcP, dstP, meta, t0, tsz):
  mesh = plsc.VectorSubcoreMesh(core_axis_name="c", subcore_axis_name="s")

  @functools.partial(
      pl.kernel,
      mesh=mesh,
      out_type=jax.ShapeDtypeStruct((_NC, _NP, _D), jnp.float32),
      scratch_types=[
          pltpu.VMEM((16,), jnp.int32),
          pltpu.VMEM((2, _CH), jnp.int32),
          pltpu.VMEM((2, _CH), jnp.int32),
          pltpu.VMEM((2, _CH, _D), jnp.float32),
          pltpu.VMEM_SHARED((_TSZ, _D), jnp.float32),
          pltpu.VMEM_SHARED((_NP, _D), jnp.float32),
          pltpu.SemaphoreType.DMA,
          pltpu.SemaphoreType.DMA,
          pltpu.SemaphoreType.DMA,
          pltpu.SemaphoreType.DMA,
      ],
  )
  def seg_kernel(h_hbm, src_hbm, dst_hbm, meta_hbm, out_hbm, meta_v, ia_v,
                 ib_v, rows_v, tab_sh, acc_sh, semia, semib, semg0, semg1):
    c = lax.axis_index("c")
    s = lax.axis_index("s")
    wid = s * _NC + c

    pltpu.sync_copy(meta_hbm, meta_v)
    mv = meta_v[pl.ds(0, 16)]
    lo8 = mv[0]
    hi = mv[1]
    lo = mv[2]
    per_tile = mv[3]

    start_t = lo8 + wid * per_tile
    cnt_t = jnp.clip(hi - start_t, 0, per_tile)
    n_t = (cnt_t + _CH - 1) // _CH

    lane = lax.iota(jnp.int32, 16)

    def idx_load(j, buf, sem):
      off = pl.multiple_of(start_t + j * _CH, 8)
      d1 = pltpu.async_copy(src_hbm.at[pl.ds(off, _CH)], buf.at[0], sem)
      d2 = pltpu.async_copy(dst_hbm.at[pl.ds(off, _CH)], buf.at[1], sem)
      return d1, d2

    def idx_wait(buf, sem):
      pltpu.make_async_copy(src_hbm.at[pl.ds(0, _CH)], buf.at[0], sem).wait()
      pltpu.make_async_copy(src_hbm.at[pl.ds(0, _CH)], buf.at[1], sem).wait()

    def transform(j, buf):
      base = start_t + j * _CH
      for q in range(_CH // 16):
        lp = base + q * 16 + lane
        sv = jnp.clip(buf[0, pl.ds(q * 16, 16)] - t0, 0, tsz - 1)
        dv = buf[1, pl.ds(q * 16, 16)]
        dv = jnp.where((lp >= lo) & (lp < hi), dv, _N)
        buf[0, pl.ds(q * 16, 16)] = sv
        buf[1, pl.ds(q * 16, 16)] = dv

    def g_start(buf, rb, sem):
      pltpu.async_copy(tab_sh.at[buf.at[0]], rows_v.at[rb], sem)

    def g_wait(buf, rb, sem):
      pltpu.make_async_copy(tab_sh.at[buf.at[0]], rows_v.at[rb], sem).wait()

    def scat(buf, rb):
      pltpu.sync_copy(rows_v.at[rb], acc_sh.at[buf.at[1]], add=True)

    @pl.when(s < _NS - 1)
    def _():
      pltpu.sync_copy(h_hbm.at[pl.ds(t0 + s * 320, 320)],
                      tab_sh.at[pl.ds(s * 320, 320)])

    @pl.when(s == _NS - 1)
    def _():
      pltpu.sync_copy(h_hbm.at[pl.ds(t0 + 4800, tsz - 4800)],
                      tab_sh.at[pl.ds(4800, tsz - 4800)])

    @pl.when(n_t >= 1)
    def _():
      d1, d2 = idx_load(0, ia_v, semia)
      d1.wait()
      d2.wait()
      transform(0, ia_v)

    @pl.when(n_t >= 2)
    def _():
      idx_load(1, ib_v, semib)

    def _zrow(i, carry):
      for k in range(_D // 16):
        rows_v[1, i, pl.ds(k * 16, 16)] = jnp.zeros((16,), jnp.float32)
      return carry
    lax.fori_loop(0, _CH, _zrow, 0)

    @pl.when(s < _NS - 1)
    def _():
      for r in range(_ARPT // _CH):
        pltpu.sync_copy(rows_v.at[1],
                        acc_sh.at[pl.ds(s * _ARPT + r * _CH, _CH)])
      pltpu.sync_copy(rows_v.at[1, pl.ds(0, _ARPT % _CH)],
                      acc_sh.at[pl.ds(s * _ARPT + (_ARPT // _CH) * _CH,
                                      _ARPT % _CH)])

    @pl.when(s == _NS - 1)
    def _():
      _LR = _NP - (_NS - 1) * _ARPT
      for r in range(_LR // _CH):
        pltpu.sync_copy(rows_v.at[1],
                        acc_sh.at[pl.ds((_NS - 1) * _ARPT + r * _CH, _CH)])
      pltpu.sync_copy(rows_v.at[1, pl.ds(0, _LR % _CH)],
                      acc_sh.at[pl.ds((_NS - 1) * _ARPT + (_LR // _CH) * _CH,
                                      _LR % _CH)])
    plsc.subcore_barrier()

    @pl.when(n_t >= 1)
    def _():
      g_start(ia_v, 0, semg0)

    def _step(j, ibj, rbj, semj, ibn, rbn, semn):
      @pl.when(j + 1 < n_t)
      def _():
        idx_wait(ibn, semn[0])
        transform(j + 1, ibn)
      g_wait(ibj, rbj, semj[1])

      @pl.when(j + 1 < n_t)
      def _():
        g_start(ibn, rbn, semn[1])
      scat(ibj, rbj)

      @pl.when(j + 2 < n_t)
      def _():
        idx_load(j + 2, ibj, semj[0])

    def _body(j, carry):
      @pl.when(j % 2 == 0)
      def _():
        _step(j, ia_v, 0, (semia, semg0), ib_v, 1, (semib, semg1))

      @pl.when(j % 2 == 1)
      def _():
        _step(j, ib_v, 1, (semib, semg1), ia_v, 0, (semia, semg0))
      return carry
    lax.fori_loop(0, n_t, _body, 0)
    plsc.subcore_barrier()

    @pl.when(s < _NS - 1)
    def _():
      pltpu.sync_copy(acc_sh.at[pl.ds(s * _ARPT, _ARPT)],
                      out_hbm.at[c, pl.ds(s * _ARPT, _ARPT)])

    @pl.when(s == _NS - 1)
    def _():
      _LR = _NP - (_NS - 1) * _ARPT
      pltpu.sync_copy(acc_sh.at[pl.ds((_NS - 1) * _ARPT, _LR)],
                      out_hbm.at[c, pl.ds((_NS - 1) * _ARPT, _LR)])

  return seg_kernel(h, srcP, dstP, meta)


def _partition_edges(edge_index):
  src = edge_index[0]
  dst = edge_index[1]
  flag = (src >= _T).astype(jnp.int32)
  c1 = jnp.cumsum(flag)
  cnt0 = _E - c1[-1]
  idx = jnp.arange(_E, dtype=jnp.int32)
  pos = jnp.where(flag == 0, idx - c1, cnt0 + c1 - 1)
  srcP = jnp.zeros((_EPAD,), jnp.int32).at[pos].set(src)
  dstP = jnp.full((_EPAD,), _N, jnp.int32).at[pos].set(dst)
  span = _NW * _CH

  def meta_for(lo, hi):
    lo8 = (lo // 8) * 8
    per_tile = ((hi - lo8 + span - 1) // span) * _CH
    z = jnp.int32(0)
    return jnp.stack([lo8, hi, lo, per_tile] + [z] * 12)

  metaA = meta_for(jnp.int32(0), cnt0)
  metaB = meta_for(cnt0, jnp.int32(_E))
  return srcP, dstP, metaA, metaB


def _seg_sum_both(h, srcP, dstP, metaA, metaB):
  aggA = _seg_sum_shard(h, srcP, dstP, metaA, 0, _T)
  aggB = _seg_sum_shard(h, srcP, dstP, metaB, _T, _N - _T)
  return aggA, aggB


_RB = 1000

_AGG_SPEC = pl.BlockSpec((2, _RB, _D), lambda i: (0, i, 0))
_W_SPEC = pl.BlockSpec((_D, _D), lambda i: (0, 0))
_B_SPEC = pl.BlockSpec((1, _D), lambda i: (0, 0))


def _mlp1(x, aggA, aggB, Wa, ba, Wb, bb):
  def body(x_ref, aa_ref, ab_ref, wa_ref, ba_ref, wb_ref, bb_ref, o_ref):
    m = x_ref[...] + (aa_ref[0] + aa_ref[1]) + (ab_ref[0] + ab_ref[1])
    m = jnp.maximum(
        jnp.dot(m, wa_ref[...], preferred_element_type=jnp.float32)
        + ba_ref[...], 0.0)
    hh = (jnp.dot(m, wb_ref[...], preferred_element_type=jnp.float32)
          + bb_ref[...])
    o_ref[...] = jnp.where(hh > 0, hh, jnp.exp(jnp.minimum(hh, 0.0)) - 1.0)

  return pl.pallas_call(
      body,
      grid=(_N // _RB,),
      in_specs=[
          pl.BlockSpec((_RB, _D), lambda i: (i, 0)),
          _AGG_SPEC, _AGG_SPEC, _W_SPEC, _B_SPEC, _W_SPEC, _B_SPEC,
      ],
      out_specs=pl.BlockSpec((_RB, _D), lambda i: (i, 0)),
      out_shape=jax.ShapeDtypeStruct((_N, _D), jnp.float32),
  )(x, aggA, aggB, Wa, ba.reshape(1, _D), Wb, bb.reshape(1, _D))


def _mlp2(h, aggA, aggB, Wa, ba, Wb, bb, Wf, bf):
  def body(h_ref, aa_ref, ab_ref, wa_ref, ba_ref, wb_ref, bb_ref, wf_ref,
           bf_ref, o_ref):
    m = h_ref[...] + (aa_ref[0] + aa_ref[1]) + (ab_ref[0] + ab_ref[1])
    m = jnp.maximum(
        jnp.dot(m, wa_ref[...], preferred_element_type=jnp.float32)
        + ba_ref[...], 0.0)
    h2 = (jnp.dot(m, wb_ref[...], preferred_element_type=jnp.float32)
          + bb_ref[...])
    h2 = jnp.where(h2 > 0, h2, jnp.exp(jnp.minimum(h2, 0.0)) - 1.0)
    z = (jnp.dot(h2, wf_ref[...], preferred_element_type=jnp.float32)
         + bf_ref[...])
    o_ref[...] = 1.0 / (1.0 + jnp.exp(-z))

  return pl.pallas_call(
      body,
      grid=(_N // _RB,),
      in_specs=[
          pl.BlockSpec((_RB, _D), lambda i: (i, 0)),
          _AGG_SPEC, _AGG_SPEC, _W_SPEC, _B_SPEC, _W_SPEC, _B_SPEC,
          pl.BlockSpec((_D, _L), lambda i: (0, 0)),
          pl.BlockSpec((1, _L), lambda i: (0, 0)),
      ],
      out_specs=pl.BlockSpec((_RB, _L), lambda i: (i, 0)),
      out_shape=jax.ShapeDtypeStruct((_N, _L), jnp.float32),
  )(h, aggA, aggB, Wa, ba.reshape(1, _D), Wb, bb.reshape(1, _D), Wf,
    bf.reshape(1, _L))


def kernel(x, edge_index, W11, b11, W12, b12, W21, b21, W22, b22, Wf, bf):
  srcP, dstP, metaA, metaB = _partition_edges(edge_index)
  a1A, a1B = _seg_sum_both(x, srcP, dstP, metaA, metaB)
  h1 = _mlp1(x, a1A, a1B, W11, b11, W12, b12)
  a2A, a2B = _seg_sum_both(h1, srcP, dstP, metaA, metaB)
  return _mlp2(h1, a2A, a2B, W21, b21, W22, b22, Wf, bf)


# speedup vs baseline: 2.9153x; 2.9153x over previous
import functools

import jax
import jax.numpy as jnp
from jax import lax
from jax.experimental import pallas as pl
from jax.experimental.pallas import tpu as pltpu
from jax.experimental.pallas import tpu_sc as plsc

_N = 10000
_D = 128
_E = 320000
_L = 16

_NC = 2
_NS = 16
_NW = _NC * _NS
_CH = 32
_T = 4992
_TSZ = 5008
_NP = 10008
_ARPT = 624
_EPAD = _E + 1536


def _seg_sum_shard(h, srcP, dstP, meta, t0, tsz):
  mesh = plsc.VectorSubcoreMesh(core_axis_name="c", subcore_axis_name="s")

  @functools.partial(
      pl.kernel,
      mesh=mesh,
      out_type=jax.ShapeDtypeStruct((_NC, _NP, _D), jnp.float32),
      scratch_types=[
          pltpu.VMEM((16,), jnp.int32),
          pltpu.VMEM((2, _CH), jnp.int32),
          pltpu.VMEM((2, _CH), jnp.int32),
          pltpu.VMEM((2, _CH, _D), jnp.float32),
          pltpu.VMEM_SHARED((_TSZ, _D), jnp.float32),
          pltpu.VMEM_SHARED((_NP, _D), jnp.float32),
          pltpu.SemaphoreType.DMA,
          pltpu.SemaphoreType.DMA,
          pltpu.SemaphoreType.DMA,
          pltpu.SemaphoreType.DMA,
      ],
  )
  def seg_kernel(h_hbm, src_hbm, dst_hbm, meta_hbm, out_hbm, meta_v, ia_v,
                 ib_v, rows_v, tab_sh, acc_sh, semia, semib, semg0, semg1):
    c = lax.axis_index("c")
    s = lax.axis_index("s")
    wid = s * _NC + c

    pltpu.sync_copy(meta_hbm, meta_v)
    mv = meta_v[pl.ds(0, 16)]
    lo8 = mv[0]
    hi = mv[1]
    lo = mv[2]
    per_tile = mv[3]

    start_t = lo8 + wid * per_tile
    cnt_t = jnp.clip(hi - start_t, 0, per_tile)
    n_t = (cnt_t + _CH - 1) // _CH

    lane = lax.iota(jnp.int32, 16)

    def idx_load(j, buf, sem):
      off = pl.multiple_of(start_t + j * _CH, 8)
      d1 = pltpu.async_copy(src_hbm.at[pl.ds(off, _CH)], buf.at[0], sem)
      d2 = pltpu.async_copy(dst_hbm.at[pl.ds(off, _CH)], buf.at[1], sem)
      return d1, d2

    def idx_wait(buf, sem):
      pltpu.make_async_copy(src_hbm.at[pl.ds(0, _CH)], buf.at[0], sem).wait()
      pltpu.make_async_copy(src_hbm.at[pl.ds(0, _CH)], buf.at[1], sem).wait()

    def transform(j, buf):
      base = start_t + j * _CH
      for q in range(_CH // 16):
        lp = base + q * 16 + lane
        sv = jnp.clip(buf[0, pl.ds(q * 16, 16)] - t0, 0, tsz - 1)
        dv = buf[1, pl.ds(q * 16, 16)]
        dv = jnp.where((lp >= lo) & (lp < hi), dv, _N)
        buf[0, pl.ds(q * 16, 16)] = sv
        buf[1, pl.ds(q * 16, 16)] = dv

    def g_start(buf, rb, sem):
      pltpu.async_copy(tab_sh.at[buf.at[0]], rows_v.at[rb], sem)

    def g_wait(buf, rb, sem):
      pltpu.make_async_copy(tab_sh.at[buf.at[0]], rows_v.at[rb], sem).wait()

    def scat(buf, rb):
      pltpu.sync_copy(rows_v.at[rb], acc_sh.at[buf.at[1]], add=True)

    @pl.when(s < _NS - 1)
    def _():
      pltpu.sync_copy(h_hbm.at[pl.ds(t0 + s * 320, 320)],
                      tab_sh.at[pl.ds(s * 320, 320)])

    @pl.when(s == _NS - 1)
    def _():
      pltpu.sync_copy(h_hbm.at[pl.ds(t0 + 4800, tsz - 4800)],
                      tab_sh.at[pl.ds(4800, tsz - 4800)])

    @pl.when(n_t >= 1)
    def _():
      d1, d2 = idx_load(0, ia_v, semia)
      d1.wait()
      d2.wait()
      transform(0, ia_v)

    @pl.when(n_t >= 2)
    def _():
      idx_load(1, ib_v, semib)

    def _zrow(i, carry):
      for k in range(_D // 16):
        rows_v[1, i, pl.ds(k * 16, 16)] = jnp.zeros((16,), jnp.float32)
      return carry
    lax.fori_loop(0, _CH, _zrow, 0)

    @pl.when(s < _NS - 1)
    def _():
      for r in range(_ARPT // _CH):
        pltpu.sync_copy(rows_v.at[1],
                        acc_sh.at[pl.ds(s * _ARPT + r * _CH, _CH)])
      pltpu.sync_copy(rows_v.at[1, pl.ds(0, _ARPT % _CH)],
                      acc_sh.at[pl.ds(s * _ARPT + (_ARPT // _CH) * _CH,
                                      _ARPT % _CH)])

    @pl.when(s == _NS - 1)
    def _():
      _LR = _NP - (_NS - 1) * _ARPT
      for r in range(_LR // _CH):
        pltpu.sync_copy(rows_v.at[1],
                        acc_sh.at[pl.ds((_NS - 1) * _ARPT + r * _CH, _CH)])
      pltpu.sync_copy(rows_v.at[1, pl.ds(0, _LR % _CH)],
                      acc_sh.at[pl.ds((_NS - 1) * _ARPT + (_LR // _CH) * _CH,
                                      _LR % _CH)])
    plsc.subcore_barrier()

    @pl.when(n_t >= 1)
    def _():
      g_start(ia_v, 0, semg0)

    def _step(j, ibj, rbj, semj, ibn, rbn, semn):
      @pl.when(j + 1 < n_t)
      def _():
        idx_wait(ibn, semn[0])
        transform(j + 1, ibn)
      g_wait(ibj, rbj, semj[1])

      @pl.when(j + 1 < n_t)
      def _():
        g_start(ibn, rbn, semn[1])
      scat(ibj, rbj)

      @pl.when(j + 2 < n_t)
      def _():
        idx_load(j + 2, ibj, semj[0])

    def _body(j, carry):
      @pl.when(j % 2 == 0)
      def _():
        _step(j, ia_v, 0, (semia, semg0), ib_v, 1, (semib, semg1))

      @pl.when(j % 2 == 1)
      def _():
        _step(j, ib_v, 1, (semib, semg1), ia_v, 0, (semia, semg0))
      return carry
    lax.fori_loop(0, n_t, _body, 0)
    plsc.subcore_barrier()

    @pl.when(s < _NS - 1)
    def _():
      pltpu.sync_copy(acc_sh.at[pl.ds(s * _ARPT, _ARPT)],
                      out_hbm.at[c, pl.ds(s * _ARPT, _ARPT)])

    @pl.when(s == _NS - 1)
    def _():
      _LR = _NP - (_NS - 1) * _ARPT
      pltpu.sync_copy(acc_sh.at[pl.ds((_NS - 1) * _ARPT, _LR)],
                      out_hbm.at[c, pl.ds((_NS - 1) * _ARPT, _LR)])

  return seg_kernel(h, srcP, dstP, meta)


def _partition_edges(edge_index):
  src = edge_index[0]
  dst = edge_index[1]
  flag = (src >= _T).astype(jnp.int32)
  cnt0 = _E - jnp.sum(flag)
  perm = jnp.argsort(flag, stable=True)
  srcP = jnp.concatenate([src[perm], jnp.zeros((_EPAD - _E,), jnp.int32)])
  dstP = jnp.concatenate([dst[perm], jnp.full((_EPAD - _E,), _N, jnp.int32)])
  span = _NW * _CH

  def meta_for(lo, hi):
    lo8 = (lo // 8) * 8
    per_tile = ((hi - lo8 + span - 1) // span) * _CH
    z = jnp.int32(0)
    return jnp.stack([lo8, hi, lo, per_tile] + [z] * 12)

  metaA = meta_for(jnp.int32(0), cnt0)
  metaB = meta_for(cnt0, jnp.int32(_E))
  return srcP, dstP, metaA, metaB


def _seg_sum_both(h, srcP, dstP, metaA, metaB):
  aggA = _seg_sum_shard(h, srcP, dstP, metaA, 0, _T)
  aggB = _seg_sum_shard(h, srcP, dstP, metaB, _T, _N - _T)
  return aggA, aggB


_RB = 1000

_AGG_SPEC = pl.BlockSpec((2, _RB, _D), lambda i: (0, i, 0))
_W_SPEC = pl.BlockSpec((_D, _D), lambda i: (0, 0))
_B_SPEC = pl.BlockSpec((1, _D), lambda i: (0, 0))


def _mlp1(x, aggA, aggB, Wa, ba, Wb, bb):
  def body(x_ref, aa_ref, ab_ref, wa_ref, ba_ref, wb_ref, bb_ref, o_ref):
    m = x_ref[...] + (aa_ref[0] + aa_ref[1]) + (ab_ref[0] + ab_ref[1])
    m = jnp.maximum(
        jnp.dot(m, wa_ref[...], preferred_element_type=jnp.float32)
        + ba_ref[...], 0.0)
    hh = (jnp.dot(m, wb_ref[...], preferred_element_type=jnp.float32)
          + bb_ref[...])
    o_ref[...] = jnp.where(hh > 0, hh, jnp.exp(jnp.minimum(hh, 0.0)) - 1.0)

  return pl.pallas_call(
      body,
      grid=(_N // _RB,),
      in_specs=[
          pl.BlockSpec((_RB, _D), lambda i: (i, 0)),
          _AGG_SPEC, _AGG_SPEC, _W_SPEC, _B_SPEC, _W_SPEC, _B_SPEC,
      ],
      out_specs=pl.BlockSpec((_RB, _D), lambda i: (i, 0)),
      out_shape=jax.ShapeDtypeStruct((_N, _D), jnp.float32),
  )(x, aggA, aggB, Wa, ba.reshape(1, _D), Wb, bb.reshape(1, _D))


def _mlp2(h, aggA, aggB, Wa, ba, Wb, bb, Wf, bf):
  def body(h_ref, aa_ref, ab_ref, wa_ref, ba_ref, wb_ref, bb_ref, wf_ref,
           bf_ref, o_ref):
    m = h_ref[...] + (aa_ref[0] + aa_ref[1]) + (ab_ref[0] + ab_ref[1])
    m = jnp.maximum(
        jnp.dot(m, wa_ref[...], preferred_element_type=jnp.float32)
        + ba_ref[...], 0.0)
    h2 = (jnp.dot(m, wb_ref[...], preferred_element_type=jnp.float32)
          + bb_ref[...])
    h2 = jnp.where(h2 > 0, h2, jnp.exp(jnp.minimum(h2, 0.0)) - 1.0)
    z = (jnp.dot(h2, wf_ref[...], preferred_element_type=jnp.float32)
         + bf_ref[...])
    o_ref[...] = 1.0 / (1.0 + jnp.exp(-z))

  return pl.pallas_call(
      body,
      grid=(_N // _RB,),
      in_specs=[
          pl.BlockSpec((_RB, _D), lambda i: (i, 0)),
          _AGG_SPEC, _AGG_SPEC, _W_SPEC, _B_SPEC, _W_SPEC, _B_SPEC,
          pl.BlockSpec((_D, _L), lambda i: (0, 0)),
          pl.BlockSpec((1, _L), lambda i: (0, 0)),
      ],
      out_specs=pl.BlockSpec((_RB, _L), lambda i: (i, 0)),
      out_shape=jax.ShapeDtypeStruct((_N, _L), jnp.float32),
  )(h, aggA, aggB, Wa, ba.reshape(1, _D), Wb, bb.reshape(1, _D), Wf,
    bf.reshape(1, _L))


def kernel(x, edge_index, W11, b11, W12, b12, W21, b21, W22, b22, Wf, bf):
  srcP, dstP, metaA, metaB = _partition_edges(edge_index)
  a1A, a1B = _seg_sum_both(x, srcP, dstP, metaA, metaB)
  h1 = _mlp1(x, a1A, a1B, W11, b11, W12, b12)
  a2A, a2B = _seg_sum_both(h1, srcP, dstP, metaA, metaB)
  return _mlp2(h1, a2A, a2B, W21, b21, W22, b22, Wf, bf)
